# baseline (speedup 1.0000x reference)
import functools

import jax
import jax.numpy as jnp
from jax import lax
from jax.experimental import pallas as pl
from jax.experimental.pallas import tpu as pltpu
from jax.experimental.pallas import tpu_sc as plsc

VOCAB_P1 = 100001
EMB = 32
LEN_SEQ = 4
MAP_H = 16
MAP_W = 16
SEQ_SIZE = 20
BATCH = 32

PAIRS = BATCH * LEN_SEQ
SEGS_PER_PAIR = MAP_H * MAP_W
IDX_PER_PAIR = SEGS_PER_PAIR * SEQ_SIZE
NW = 32
PAIRS_PER_W = PAIRS // NW
IDX_ROW = 128
ROWS_PER_CHUNK = 640
SEGS_PER_CHUNK = ROWS_PER_CHUNK // SEQ_SIZE
CHUNKS = IDX_PER_PAIR // ROWS_PER_CHUNK
GATHERS_PER_CHUNK = ROWS_PER_CHUNK // IDX_ROW


def _tree_sum(vals):
    while len(vals) > 1:
        nxt = [vals[i] + vals[i + 1] for i in range(0, len(vals) - 1, 2)]
        if len(vals) % 2:
            nxt.append(vals[-1])
        vals = nxt
    return vals[0]


def _sc_body(x_hbm, emb_hbm, out_hbm, idx_v, rows_v, acc_v, sem):
    wid = lax.axis_index("s") * 2 + lax.axis_index("c")
    gchunks = PAIRS_PER_W * CHUNKS

    def chunk_copies(g, buf):
        cps = []
        for j in range(GATHERS_PER_CHUNK):
            cps.append(
                pltpu.make_async_copy(
                    emb_hbm.at[
                        idx_v.at[
                            g >> 3,
                            pl.ds(((g & 7) * GATHERS_PER_CHUNK + j) * IDX_ROW, IDX_ROW),
                        ]
                    ],
                    rows_v.at[buf, pl.ds(j * IDX_ROW, IDX_ROW)],
                    sem,
                )
            )
        return cps

    pltpu.sync_copy(x_hbm.at[pl.ds(wid * PAIRS_PER_W, PAIRS_PER_W)], idx_v)
    for cp in chunk_copies(0, 0):
        cp.start()

    def g_body(g, _):
        buf = g & 1
        pltpu.make_async_copy(
            emb_hbm.at[pl.ds(0, ROWS_PER_CHUNK)], rows_v.at[buf], sem
        ).wait()

        @pl.when(g + 1 < gchunks)
        def _():
            for cp in chunk_copies(g + 1, 1 - buf):
                cp.start()

        iota = lax.iota(jnp.int32, 16)

        @plsc.parallel_loop(0, SEGS_PER_CHUNK, unroll=4)
        def seg_body(s):
            base = s * SEQ_SIZE
            rows = [rows_v[buf, base + k, :] for k in range(SEQ_SIZE)]
            l1 = [rows[2 * i] + rows[2 * i + 1] for i in range(10)]
            l2 = [l1[2 * i] + l1[2 * i + 1] for i in range(5)]
            parts = [
                plsc.unpack(p, format=plsc.PackFormat.INTERLEAVED) for p in l2
            ]
            a0 = _tree_sum([p[0] for p in parts])
            a1 = _tree_sum([p[1] for p in parts])
            seg = (g & 7) * SEGS_PER_CHUNK + s
            rowv = jnp.full((16,), seg, jnp.int32)
            plsc.store_scatter(acc_v, [rowv, iota * 2], a0)
            plsc.store_scatter(acc_v, [rowv, iota * 2 + 1], a1)

        @pl.when((g & 7) == 7)
        def _():
            pi = g >> 3
            pltpu.sync_copy(acc_v, out_hbm.at[wid, :, pl.ds(pi * EMB, EMB)])

        return 0

    lax.fori_loop(0, gchunks, g_body, 0)


@functools.partial(jax.jit, static_argnames=())
def kernel(x, embeddings):
    x = x.astype(jnp.int32).reshape(PAIRS, IDX_PER_PAIR)
    embeddings = embeddings.astype(jnp.bfloat16)
    mesh = plsc.VectorSubcoreMesh(core_axis_name="c", subcore_axis_name="s")
    out = pl.kernel(
        _sc_body,
        mesh=mesh,
        compiler_params=pltpu.CompilerParams(
            needs_layout_passes=False, use_tc_tiling_on_sc=False
        ),
        out_type=jax.ShapeDtypeStruct(
            (BATCH, SEGS_PER_PAIR, LEN_SEQ * EMB), jnp.float32
        ),
        scratch_types=[
            pltpu.VMEM((PAIRS_PER_W, IDX_PER_PAIR), jnp.int32),
            pltpu.VMEM((2, ROWS_PER_CHUNK, EMB), jnp.bfloat16),
            pltpu.VMEM((SEGS_PER_PAIR, EMB), jnp.float32),
            pltpu.SemaphoreType.DMA,
        ],
    )(x, embeddings)
    return out.reshape(BATCH, MAP_H, MAP_W, LEN_SEQ * EMB).transpose(0, 3, 1, 2)


# speedup vs baseline: 1.2174x; 1.2174x over previous
import functools

import jax
import jax.numpy as jnp
from jax import lax
from jax.experimental import pallas as pl
from jax.experimental.pallas import tpu as pltpu
from jax.experimental.pallas import tpu_sc as plsc

VOCAB_P1 = 100001
EMB = 32
LEN_SEQ = 4
MAP_H = 16
MAP_W = 16
SEQ_SIZE = 20
BATCH = 32

PAIRS = BATCH * LEN_SEQ
SEGS_PER_PAIR = MAP_H * MAP_W
IDX_PER_PAIR = SEGS_PER_PAIR * SEQ_SIZE
NW = 32
PAIRS_PER_W = PAIRS // NW
IDX_ROW = 128
ROWS_PER_CHUNK = 640
SEGS_PER_CHUNK = ROWS_PER_CHUNK // SEQ_SIZE
CHUNKS = IDX_PER_PAIR // ROWS_PER_CHUNK
GATHERS_PER_CHUNK = ROWS_PER_CHUNK // IDX_ROW


def _tree_sum(vals):
    while len(vals) > 1:
        nxt = [vals[i] + vals[i + 1] for i in range(0, len(vals) - 1, 2)]
        if len(vals) % 2:
            nxt.append(vals[-1])
        vals = nxt
    return vals[0]


def _sc_body(x_hbm, emb_hbm, out_hbm, idx_v, rows_v, acc_v, sem):
    wid = lax.axis_index("s") * 2 + lax.axis_index("c")
    gchunks = PAIRS_PER_W * CHUNKS

    def chunk_copies(g, buf):
        cps = []
        for j in range(GATHERS_PER_CHUNK):
            cps.append(
                pltpu.make_async_copy(
                    emb_hbm.at[
                        idx_v.at[
                            g >> 3,
                            pl.ds(((g & 7) * GATHERS_PER_CHUNK + j) * IDX_ROW, IDX_ROW),
                        ]
                    ],
                    rows_v.at[buf, pl.ds(j * IDX_ROW, IDX_ROW)],
                    sem,
                )
            )
        return cps

    pltpu.sync_copy(x_hbm.at[pl.ds(wid * PAIRS_PER_W, PAIRS_PER_W)], idx_v)
    for cp in chunk_copies(0, 0):
        cp.start()

    def g_body(g, _):
        buf = g & 1
        pltpu.make_async_copy(
            emb_hbm.at[pl.ds(0, ROWS_PER_CHUNK)], rows_v.at[buf], sem
        ).wait()

        @pl.when(g + 1 < gchunks)
        def _():
            for cp in chunk_copies(g + 1, 1 - buf):
                cp.start()

        @plsc.parallel_loop(0, SEGS_PER_CHUNK, unroll=4)
        def seg_body(s):
            base = s * SEQ_SIZE
            a0 = _tree_sum(
                [rows_v[buf, base + k, pl.ds(0, 16)] for k in range(SEQ_SIZE)]
            )
            a1 = _tree_sum(
                [rows_v[buf, base + k, pl.ds(16, 16)] for k in range(SEQ_SIZE)]
            )
            seg = (g & 7) * SEGS_PER_CHUNK + s
            acc_v[seg, pl.ds(0, 16)] = a0
            acc_v[seg, pl.ds(16, 16)] = a1

        @pl.when((g & 7) == 7)
        def _():
            pi = g >> 3
            pltpu.sync_copy(acc_v, out_hbm.at[wid, :, pl.ds(pi * EMB, EMB)])

        return 0

    lax.fori_loop(0, gchunks, g_body, 0)


@functools.partial(jax.jit, static_argnames=())
def kernel(x, embeddings):
    x = x.astype(jnp.int32).reshape(PAIRS, IDX_PER_PAIR)
    mesh = plsc.VectorSubcoreMesh(core_axis_name="c", subcore_axis_name="s")
    out = pl.kernel(
        _sc_body,
        mesh=mesh,
        compiler_params=pltpu.CompilerParams(
            needs_layout_passes=False, use_tc_tiling_on_sc=False
        ),
        out_type=jax.ShapeDtypeStruct(
            (BATCH, SEGS_PER_PAIR, LEN_SEQ * EMB), jnp.float32
        ),
        scratch_types=[
            pltpu.VMEM((PAIRS_PER_W, IDX_PER_PAIR), jnp.int32),
            pltpu.VMEM((2, ROWS_PER_CHUNK, EMB), jnp.float32),
            pltpu.VMEM((SEGS_PER_PAIR, EMB), jnp.float32),
            pltpu.SemaphoreType.DMA,
        ],
    )(x, embeddings)
    return out.reshape(BATCH, MAP_H, MAP_W, LEN_SEQ * EMB).transpose(0, 3, 1, 2)
